# Initial kernel scaffold; baseline (speedup 1.0000x reference)
import jax
import jax.numpy as jnp
from jax.experimental import pallas as pl


def kernel(x, edge_index, batch, W0a, b0a, W0b, b0b, g0, be0, W1a, b1a, W1b, b1b, g1, be1, W2a, b2a, W2b, b2b, g2, be2):
    raise NotImplementedError("write your pallas kernel here")


# speedup vs baseline: 6.1919x; 6.1919x over previous
import jax
import jax.numpy as jnp
from jax import lax
from jax.experimental import pallas as pl
from jax.experimental.pallas import tpu as pltpu
from jax.experimental.pallas import tpu_sc as plsc

N = 10000
E = 320000
D = 128
CHUNK = 128
NUM_CHUNKS = E // CHUNK
NC = 2
NS = 16
NW = NC * NS
ROWS_PER_TILE = 624
TAIL_ROWS = N - NS * ROWS_PER_TILE


def _sc_agg_body(h_hbm, src_hbm, dst_hbm, out_hbm, sidx, didx, rows, acc, sem):
    cid = lax.axis_index("c")
    sid = lax.axis_index("s")
    wid = sid * NC + cid

    r0 = sid * ROWS_PER_TILE
    pltpu.sync_copy(h_hbm.at[pl.ds(r0, ROWS_PER_TILE)],
                    acc.at[pl.ds(r0, ROWS_PER_TILE)])

    @pl.when(sid == NS - 1)
    def _():
        pltpu.sync_copy(h_hbm.at[pl.ds(NS * ROWS_PER_TILE, TAIL_ROWS)],
                        acc.at[pl.ds(NS * ROWS_PER_TILE, TAIL_ROWS)])

    plsc.subcore_barrier()

    nchunks = (NUM_CHUNKS - wid + NW - 1) // NW

    def body(j, _):
        off = (wid + j * NW) * CHUNK
        pltpu.sync_copy(src_hbm.at[pl.ds(off, CHUNK)], sidx)
        pltpu.sync_copy(dst_hbm.at[pl.ds(off, CHUNK)], didx)
        pltpu.async_copy(h_hbm.at[sidx], rows, sem).wait()
        pltpu.sync_copy(rows, acc.at[didx], add=True)
        return _

    lax.fori_loop(0, nchunks, body, 0)
    plsc.subcore_barrier()

    pltpu.sync_copy(acc.at[pl.ds(r0, ROWS_PER_TILE)],
                    out_hbm.at[cid, pl.ds(r0, ROWS_PER_TILE)])

    @pl.when(sid == NS - 1)
    def _():
        pltpu.sync_copy(acc.at[pl.ds(NS * ROWS_PER_TILE, TAIL_ROWS)],
                        out_hbm.at[cid, pl.ds(NS * ROWS_PER_TILE, TAIL_ROWS)])


def _sc_agg(h, src, dst):
    mesh = plsc.VectorSubcoreMesh(core_axis_name="c", subcore_axis_name="s")
    return pl.kernel(
        _sc_agg_body,
        out_type=jax.ShapeDtypeStruct((NC, N, D), jnp.float32),
        mesh=mesh,
        scratch_types=[
            pltpu.VMEM((CHUNK,), jnp.int32),
            pltpu.VMEM((CHUNK,), jnp.int32),
            pltpu.VMEM((CHUNK, D), jnp.float32),
            pltpu.VMEM_SHARED((N, D), jnp.float32),
            pltpu.SemaphoreType.DMA,
        ],
    )(h, src, dst)


def _tc_layer_body(h_ref, p_ref, wa_ref, ba_ref, wb_ref, bb_ref, g_ref,
                   be_ref, out_ref):
    h = p_ref[0] + p_ref[1] - h_ref[...]
    h = lax.dot_general(h, wa_ref[...], (((1,), (1,)), ((), ())),
                        preferred_element_type=jnp.float32)
    h = jnp.maximum(h + ba_ref[...], 0.0)
    h = lax.dot_general(h, wb_ref[...], (((1,), (1,)), ((), ())),
                        preferred_element_type=jnp.float32)
    h = jnp.maximum(h + bb_ref[...], 0.0)
    mean = jnp.mean(h, axis=0, keepdims=True)
    c = h - mean
    var = jnp.mean(c * c, axis=0, keepdims=True)
    out_ref[...] = g_ref[...] * c * lax.rsqrt(var + 1e-5) + be_ref[...]


def _tc_layer(h, p, Wa, ba, Wb, bb, g, be):
    return pl.pallas_call(
        _tc_layer_body,
        out_shape=jax.ShapeDtypeStruct((N, D), jnp.float32),
    )(h, p, Wa, ba, Wb, bb, g, be)


def kernel(x, edge_index, batch, W0a, b0a, W0b, b0b, g0, be0, W1a, b1a,
           W1b, b1b, g1, be1, W2a, b2a, W2b, b2b, g2, be2):
    params = [
        (W0a, b0a, W0b, b0b, g0, be0),
        (W1a, b1a, W1b, b1b, g1, be1),
        (W2a, b2a, W2b, b2b, g2, be2),
    ]
    src = edge_index[0]
    dst = edge_index[1]
    h = x
    xs = []
    for (Wa, ba, Wb, bb, g, be) in params:
        p = _sc_agg(h, src, dst)
        h = _tc_layer(h, p, Wa, ba, Wb, bb, g, be)
        xs.append(h)
    return jnp.concatenate(xs, axis=1)
